# baseline (speedup 1.0000x reference)
import jax
import jax.numpy as jnp
from jax import lax
from jax.experimental import pallas as pl
from jax.experimental.pallas import tpu as pltpu

B = 1024
V = 100000
TR, TL = 8, 128
NROW = B * V // TL
NBLK = B // TL
UNROLL = 16


NSEM = 4


def _body(flat2, inp_s, inp_v, mask_v, out_v, gathered, sems):
    for b in range(NBLK):
        def issue(j, carry, b=b):
            for u in range(UNROLL):
                jj = b * TL + j * UNROLL + u
                v = inp_s[jj]
                row = (lax.shift_left(lax.bitwise_and(v, ~(TR - 1)), 3)
                       + lax.bitwise_and(v, TR - 1) + b * TR)
                pltpu.make_async_copy(
                    flat2.at[pl.ds(row, 1), :],
                    gathered.at[pl.ds(jj, 1), :],
                    sems.at[u % NSEM],
                ).start()
            return carry

        lax.fori_loop(0, TL // UNROLL, issue, 0)

    for k in range(NSEM):
        for _ in range(B // TL // NSEM * 2):
            pltpu.make_async_copy(
                flat2.at[pl.ds(0, TL // 2), :],
                gathered.at[pl.ds(0, TL // 2), :],
                sems.at[k],
            ).wait()

    i_vec = lax.broadcasted_iota(jnp.int32, (B,), 0)
    lane = lax.bitwise_and(i_vec, TL - 1)
    sel = jnp.where(
        lane[:, None] == lax.broadcasted_iota(jnp.int32, (B, TL), 1),
        gathered[...], 0.0)
    vals = jnp.sum(sel, axis=1)
    out_v[...] = -vals * mask_v[...]


_gather_tc = pl.pallas_call(
    _body,
    out_shape=jax.ShapeDtypeStruct((B,), jnp.float32),
    in_specs=[
        pl.BlockSpec(memory_space=pl.ANY),
        pl.BlockSpec(memory_space=pltpu.SMEM),
        pl.BlockSpec(memory_space=pltpu.VMEM),
        pl.BlockSpec(memory_space=pltpu.VMEM),
    ],
    out_specs=pl.BlockSpec(memory_space=pltpu.VMEM),
    scratch_shapes=[
        pltpu.VMEM((B, TL), jnp.float32),
        pltpu.SemaphoreType.DMA((NSEM,)),
    ],
)


@jax.jit
def kernel(cost, inputs, mask):
    flat2 = (cost.T.reshape(V // TR, TR, B // TL, TL)
             .transpose(0, 2, 1, 3).reshape(NROW, TL))
    inputs = inputs.astype(jnp.int32)
    return _gather_tc(flat2, inputs, inputs, mask)


# speedup vs baseline: 1.0177x; 1.0177x over previous
import jax
import jax.numpy as jnp
from jax import lax
from jax.experimental import pallas as pl
from jax.experimental.pallas import tpu as pltpu

B = 1024
V = 100000
TR, TL = 8, 128
NROW = B * V // TL
NBLK = B // TL
UNROLL = 8


def _body(flat2, inp_s, inp_v, mask_v, out_v, gathered, sem):
    for b in range(NBLK):
        def issue(j, carry, b=b):
            for u in range(UNROLL):
                jj = b * TL + j * UNROLL + u
                v = inp_s[jj]
                row = (lax.shift_left(lax.bitwise_and(v, ~(TR - 1)), 3)
                       + lax.bitwise_and(v, TR - 1) + b * TR)
                pltpu.make_async_copy(
                    flat2.at[pl.ds(row, 1), :],
                    gathered.at[pl.ds(jj, 1), :],
                    sem,
                ).start()
            return carry

        lax.fori_loop(0, TL // UNROLL, issue, 0)

    for _ in range(NBLK):
        pltpu.make_async_copy(
            flat2.at[pl.ds(0, TL), :],
            gathered.at[pl.ds(0, TL), :],
            sem,
        ).wait()

    i_vec = lax.broadcasted_iota(jnp.int32, (B,), 0)
    lane = lax.bitwise_and(i_vec, TL - 1)
    sel = jnp.where(
        lane[:, None] == lax.broadcasted_iota(jnp.int32, (B, TL), 1),
        gathered[...], 0.0)
    vals = jax.lax.dot_general(
        sel, jnp.ones((TL,), jnp.float32), (((1,), (0,)), ((), ())),
        preferred_element_type=jnp.float32)
    out_v[...] = -vals * mask_v[...]


_gather_tc = pl.pallas_call(
    _body,
    out_shape=jax.ShapeDtypeStruct((B,), jnp.float32),
    in_specs=[
        pl.BlockSpec(memory_space=pl.ANY),
        pl.BlockSpec(memory_space=pltpu.SMEM),
        pl.BlockSpec(memory_space=pltpu.VMEM),
        pl.BlockSpec(memory_space=pltpu.VMEM),
    ],
    out_specs=pl.BlockSpec(memory_space=pltpu.VMEM),
    scratch_shapes=[
        pltpu.VMEM((B, TL), jnp.float32),
        pltpu.SemaphoreType.DMA,
    ],
)


@jax.jit
def kernel(cost, inputs, mask):
    flat2 = (cost.T.reshape(V // TR, TR, B // TL, TL)
             .transpose(0, 2, 1, 3).reshape(NROW, TL))
    inputs = inputs.astype(jnp.int32)
    return _gather_tc(flat2, inputs, inputs, mask)
